# baseline (speedup 1.0000x reference)
import dataclasses
import functools

import jax
import jax.numpy as jnp
from jax import lax
from jax.experimental import pallas as pl
from jax.experimental.pallas import tpu as pltpu
from jax.experimental.pallas import tpu_sc as plsc

N_NODES = 10000
N_EDGES = 320000
DIM = 128

NC = 2
NS = 16
NW = NC * NS
LANES = 16

N_PAD = 10240
CHUNK = 128
CH_PER_TILE = 80
E_TILE = CHUNK * CH_PER_TILE
E_PAD = E_TILE * NW

_vmesh = plsc.VectorSubcoreMesh(core_axis_name="c", subcore_axis_name="s")

_sc_params = pltpu.CompilerParams()
if "needs_layout_passes" in pltpu.CompilerParams.__dataclass_fields__:
    _sc_params = dataclasses.replace(_sc_params, needs_layout_passes=False)


def _deg_hist_kernel(dst_hbm, out_hbm, dst_v, hist_v, sem):
    wid = lax.axis_index("c") * NS + lax.axis_index("s")

    @pl.loop(0, N_PAD, step=LANES)
    def _zero(i):
        hist_v[pl.ds(i, LANES)] = jnp.zeros((LANES,), jnp.float32)

    pltpu.async_copy(
        dst_hbm.at[pl.ds(wid * CH_PER_TILE, CH_PER_TILE)], dst_v, sem).wait()

    ones = jnp.ones((LANES,), jnp.float32)

    @pl.loop(0, CH_PER_TILE)
    def _row(r):
        @pl.loop(0, CHUNK, step=LANES)
        def _col(c):
            idx = dst_v[r, pl.ds(c, LANES)]
            plsc.addupdate_scatter(hist_v, [idx], ones)

    pltpu.sync_copy(hist_v, out_hbm.at[wid])


def _deg_hist(dst2d):
    k = pl.kernel(
        _deg_hist_kernel,
        out_type=jax.ShapeDtypeStruct((NW, N_PAD), jnp.float32),
        mesh=_vmesh,
        scratch_types=[
            pltpu.VMEM((CH_PER_TILE, CHUNK), jnp.int32),
            pltpu.VMEM((N_PAD,), jnp.float32),
            pltpu.SemaphoreType.DMA,
        ],
        compiler_params=_sc_params,
    )
    return k(dst2d)


def _prep_kernel(x_ref, w_ref, parts_ref, y_ref, disb_ref):
    xw = jnp.dot(x_ref[...], w_ref[...], preferred_element_type=jnp.float32)
    deg = jnp.sum(parts_ref[...], axis=1, keepdims=True) + 1.0
    dis = lax.rsqrt(deg)
    y_ref[...] = xw * dis
    disb_ref[...] = jnp.broadcast_to(dis, xw.shape)


def _prep(x, w, parts_t):
    blk = 1000
    grid = N_NODES // blk
    return pl.pallas_call(
        _prep_kernel,
        grid=(grid,),
        in_specs=[
            pl.BlockSpec((blk, DIM), lambda i: (i, 0)),
            pl.BlockSpec((DIM, DIM), lambda i: (0, 0)),
            pl.BlockSpec((blk, NW), lambda i: (i, 0)),
        ],
        out_specs=[
            pl.BlockSpec((blk, DIM), lambda i: (i, 0)),
            pl.BlockSpec((blk, DIM), lambda i: (i, 0)),
        ],
        out_shape=[
            jax.ShapeDtypeStruct((N_NODES, DIM), jnp.float32),
            jax.ShapeDtypeStruct((N_NODES, DIM), jnp.float32),
        ],
    )(x, w, parts_t)


ROWS_PER_TILE = N_PAD // NS
HALF = CH_PER_TILE // 2


def _scatter_kernel(y_hbm, src_hbm, dst_hbm, zeros_hbm, out_hbm,
                    src_v, dst_v, buf0, buf1, acc_sh, s0, s1, sz):
    cid = lax.axis_index("c")
    sid = lax.axis_index("s")
    wid = cid * NS + sid
    base = wid * CH_PER_TILE

    pltpu.async_copy(
        zeros_hbm.at[pl.ds(sid * ROWS_PER_TILE, ROWS_PER_TILE)],
        acc_sh.at[pl.ds(sid * ROWS_PER_TILE, ROWS_PER_TILE)], sz).wait()
    plsc.subcore_barrier()

    for h in range(2):
        hb = base + h * HALF
        pltpu.async_copy(src_hbm.at[pl.ds(hb, HALF)], src_v, s0)
        pltpu.async_copy(dst_hbm.at[pl.ds(hb, HALF)], dst_v, s1)
        pltpu.make_async_copy(src_hbm.at[pl.ds(hb, HALF)], src_v, s0).wait()
        pltpu.make_async_copy(dst_hbm.at[pl.ds(hb, HALF)], dst_v, s1).wait()

        pltpu.async_copy(y_hbm.at[src_v.at[0]], buf0, s0)

        @pl.loop(0, HALF // 2)
        def _main(i):
            j0 = 2 * i
            pltpu.async_copy(y_hbm.at[src_v.at[j0 + 1]], buf1, s1)
            pltpu.make_async_copy(y_hbm.at[src_v.at[0]], buf0, s0).wait()
            pltpu.sync_copy(buf0, acc_sh.at[dst_v.at[j0]], add=True)
            nxt = jnp.minimum(j0 + 2, HALF - 1)
            pltpu.async_copy(y_hbm.at[src_v.at[nxt]], buf0, s0)
            pltpu.make_async_copy(y_hbm.at[src_v.at[0]], buf1, s1).wait()
            pltpu.sync_copy(buf1, acc_sh.at[dst_v.at[j0 + 1]], add=True)

        pltpu.make_async_copy(y_hbm.at[src_v.at[0]], buf0, s0).wait()

    plsc.subcore_barrier()
    pltpu.sync_copy(
        acc_sh.at[pl.ds(sid * ROWS_PER_TILE, ROWS_PER_TILE)],
        out_hbm.at[cid, pl.ds(sid * ROWS_PER_TILE, ROWS_PER_TILE)])


def _scatter(y, src2d, dst2d, zeros_hbm):
    k = pl.kernel(
        _scatter_kernel,
        out_type=jax.ShapeDtypeStruct((NC, N_PAD, DIM), jnp.float32),
        mesh=_vmesh,
        scratch_types=[
            pltpu.VMEM((HALF, CHUNK), jnp.int32),
            pltpu.VMEM((HALF, CHUNK), jnp.int32),
            pltpu.VMEM((CHUNK, DIM), jnp.float32),
            pltpu.VMEM((CHUNK, DIM), jnp.float32),
            pltpu.VMEM_SHARED((N_PAD, DIM), jnp.float32),
            pltpu.SemaphoreType.DMA,
            pltpu.SemaphoreType.DMA,
            pltpu.SemaphoreType.DMA,
        ],
        compiler_params=_sc_params,
    )
    return k(y, src2d, dst2d, zeros_hbm)


def _combine_kernel(a0_ref, a1_ref, y_ref, disb_ref, b_ref, o_ref):
    o_ref[...] = disb_ref[...] * (a0_ref[...] + a1_ref[...] + y_ref[...]) + b_ref[...]


def _combine(acc, y, disb, b2d):
    blk = 1000
    grid = N_NODES // blk
    row = pl.BlockSpec((blk, DIM), lambda i: (i, 0))
    return pl.pallas_call(
        _combine_kernel,
        grid=(grid,),
        in_specs=[row, row, row, row, pl.BlockSpec((1, DIM), lambda i: (0, 0))],
        out_specs=row,
        out_shape=jax.ShapeDtypeStruct((N_NODES, DIM), jnp.float32),
    )(acc[0], acc[1], y, disb, b2d)


@jax.jit
def kernel(x, edge_index, W, b):
    src = edge_index[0].astype(jnp.int32)
    dst = edge_index[1].astype(jnp.int32)
    n_fill = E_PAD - N_EDGES
    fill_dst = N_NODES + (jnp.arange(n_fill, dtype=jnp.int32) % (N_PAD - N_NODES))
    fill_src = jnp.zeros((n_fill,), jnp.int32)
    src2d = jnp.concatenate([src, fill_src]).reshape(NW * CH_PER_TILE, CHUNK)
    dst2d = jnp.concatenate([dst, fill_dst]).reshape(NW * CH_PER_TILE, CHUNK)

    parts = _deg_hist(dst2d)
    y, disb = _prep(x, W, parts.T[:N_NODES])
    acc = _scatter(y, src2d, dst2d, jnp.zeros((N_PAD, DIM), jnp.float32))
    out = _combine(acc, y, disb, b.reshape(1, DIM))
    return out


# speedup vs baseline: 2.5274x; 2.5274x over previous
import dataclasses
import functools

import jax
import jax.numpy as jnp
from jax import lax
from jax.experimental import pallas as pl
from jax.experimental.pallas import tpu as pltpu
from jax.experimental.pallas import tpu_sc as plsc

N_NODES = 10000
N_EDGES = 320000
DIM = 128

NC = 2
NS = 16
NW = NC * NS
LANES = 16

N_PAD = 10240
CHUNK = 128
CH_PER_TILE = 80
E_TILE = CHUNK * CH_PER_TILE
E_PAD = E_TILE * NW

_vmesh = plsc.VectorSubcoreMesh(core_axis_name="c", subcore_axis_name="s")

_sc_params = pltpu.CompilerParams()
if "needs_layout_passes" in pltpu.CompilerParams.__dataclass_fields__:
    _sc_params = dataclasses.replace(_sc_params, needs_layout_passes=False)


def _deg_hist_kernel(dst_hbm, out_hbm, dst_v, hist_v, sem):
    wid = lax.axis_index("c") * NS + lax.axis_index("s")

    @pl.loop(0, N_PAD, step=LANES)
    def _zero(i):
        hist_v[pl.ds(i, LANES)] = jnp.zeros((LANES,), jnp.float32)

    pltpu.async_copy(
        dst_hbm.at[pl.ds(wid * CH_PER_TILE, CH_PER_TILE)], dst_v, sem).wait()

    ones = jnp.ones((LANES,), jnp.float32)

    @pl.loop(0, CH_PER_TILE)
    def _row(r):
        @pl.loop(0, CHUNK, step=LANES)
        def _col(c):
            idx = dst_v[r, pl.ds(c, LANES)]
            plsc.addupdate_scatter(hist_v, [idx], ones)

    pltpu.sync_copy(hist_v, out_hbm.at[wid])


def _deg_hist(dst2d):
    k = pl.kernel(
        _deg_hist_kernel,
        out_type=jax.ShapeDtypeStruct((NW, N_PAD), jnp.float32),
        mesh=_vmesh,
        scratch_types=[
            pltpu.VMEM((CH_PER_TILE, CHUNK), jnp.int32),
            pltpu.VMEM((N_PAD,), jnp.float32),
            pltpu.SemaphoreType.DMA,
        ],
        compiler_params=_sc_params,
    )
    return k(dst2d)


def _prep_kernel(x_ref, w_ref, parts_ref, y_ref, disb_ref):
    xw = jnp.dot(x_ref[...], w_ref[...], preferred_element_type=jnp.float32)
    deg = jnp.sum(parts_ref[...], axis=1, keepdims=True) + 1.0
    dis = lax.rsqrt(deg)
    y_ref[...] = xw * dis
    disb_ref[...] = jnp.broadcast_to(dis, xw.shape)


def _prep(x, w, parts_t):
    blk = 1000
    grid = N_NODES // blk
    return pl.pallas_call(
        _prep_kernel,
        grid=(grid,),
        in_specs=[
            pl.BlockSpec((blk, DIM), lambda i: (i, 0)),
            pl.BlockSpec((DIM, DIM), lambda i: (0, 0)),
            pl.BlockSpec((blk, NW), lambda i: (i, 0)),
        ],
        out_specs=[
            pl.BlockSpec((blk, DIM), lambda i: (i, 0)),
            pl.BlockSpec((blk, DIM), lambda i: (i, 0)),
        ],
        out_shape=[
            jax.ShapeDtypeStruct((N_NODES, DIM), jnp.float32),
            jax.ShapeDtypeStruct((N_NODES, DIM), jnp.float32),
        ],
    )(x, w, parts_t)


ROWS_PER_TILE = N_PAD // NS
HALF = CH_PER_TILE // 2


def _scatter_kernel(y_hbm, src_hbm, dst_hbm, zeros_hbm, out_hbm,
                    src_v, dst_v, buf0, buf1, acc_sh, s0, s1, sz):
    cid = lax.axis_index("c")
    sid = lax.axis_index("s")
    wid = cid * NS + sid
    base = wid * CH_PER_TILE

    pltpu.async_copy(
        zeros_hbm.at[pl.ds(sid * ROWS_PER_TILE, ROWS_PER_TILE)],
        acc_sh.at[pl.ds(sid * ROWS_PER_TILE, ROWS_PER_TILE)], sz).wait()
    plsc.subcore_barrier()

    for h in range(2):
        hb = base + h * HALF
        pltpu.async_copy(src_hbm.at[pl.ds(hb, HALF)], src_v, s0)
        pltpu.async_copy(dst_hbm.at[pl.ds(hb, HALF)], dst_v, s1)
        pltpu.make_async_copy(src_hbm.at[pl.ds(hb, HALF)], src_v, s0).wait()
        pltpu.make_async_copy(dst_hbm.at[pl.ds(hb, HALF)], dst_v, s1).wait()

        pltpu.async_copy(y_hbm.at[src_v.at[0]], buf0, s0)

        @pl.loop(0, HALF // 2)
        def _main(i):
            j0 = 2 * i
            pltpu.async_copy(y_hbm.at[src_v.at[j0 + 1]], buf1, s1)
            pltpu.make_async_copy(y_hbm.at[src_v.at[0]], buf0, s0).wait()
            pltpu.sync_copy(buf0, acc_sh.at[dst_v.at[j0]], add=True)
            nxt = jnp.minimum(j0 + 2, HALF - 1)
            pltpu.async_copy(y_hbm.at[src_v.at[nxt]], buf0, s0)
            pltpu.make_async_copy(y_hbm.at[src_v.at[0]], buf1, s1).wait()
            pltpu.sync_copy(buf1, acc_sh.at[dst_v.at[j0 + 1]], add=True)

        pltpu.make_async_copy(y_hbm.at[src_v.at[0]], buf0, s0).wait()

    plsc.subcore_barrier()
    pltpu.sync_copy(
        acc_sh.at[pl.ds(sid * ROWS_PER_TILE, ROWS_PER_TILE)],
        out_hbm.at[cid, pl.ds(sid * ROWS_PER_TILE, ROWS_PER_TILE)])


def _scatter(y, src2d, dst2d, zeros_hbm):
    k = pl.kernel(
        _scatter_kernel,
        out_type=jax.ShapeDtypeStruct((NC, N_PAD, DIM), jnp.float32),
        mesh=_vmesh,
        scratch_types=[
            pltpu.VMEM((HALF, CHUNK), jnp.int32),
            pltpu.VMEM((HALF, CHUNK), jnp.int32),
            pltpu.VMEM((CHUNK, DIM), jnp.float32),
            pltpu.VMEM((CHUNK, DIM), jnp.float32),
            pltpu.VMEM_SHARED((N_PAD, DIM), jnp.float32),
            pltpu.SemaphoreType.DMA,
            pltpu.SemaphoreType.DMA,
            pltpu.SemaphoreType.DMA,
        ],
        compiler_params=_sc_params,
    )
    return k(y, src2d, dst2d, zeros_hbm)


def _combine_kernel(a0_ref, a1_ref, y_ref, disb_ref, b_ref, o_ref):
    o_ref[...] = disb_ref[...] * (a0_ref[...] + a1_ref[...] + y_ref[...]) + b_ref[...]


def _combine(acc, y, disb, b2d):
    blk = 1000
    grid = N_NODES // blk
    row = pl.BlockSpec((blk, DIM), lambda i: (i, 0))
    return pl.pallas_call(
        _combine_kernel,
        grid=(grid,),
        in_specs=[row, row, row, row, pl.BlockSpec((1, DIM), lambda i: (0, 0))],
        out_specs=row,
        out_shape=jax.ShapeDtypeStruct((N_NODES, DIM), jnp.float32),
    )(acc[0], acc[1], y, disb, b2d)


@jax.jit
def kernel(x, edge_index, W, b):
    src = edge_index[0].astype(jnp.int32)
    dst = edge_index[1].astype(jnp.int32)
    n_fill = E_PAD - N_EDGES
    fill_dst = N_NODES + (jnp.arange(n_fill, dtype=jnp.int32) % (N_PAD - N_NODES))
    fill_src = jnp.arange(n_fill, dtype=jnp.int32) % N_NODES
    src2d = jnp.concatenate([src, fill_src]).reshape(NW * CH_PER_TILE, CHUNK)
    dst2d = jnp.concatenate([dst, fill_dst]).reshape(NW * CH_PER_TILE, CHUNK)

    parts = _deg_hist(dst2d)
    y, disb = _prep(x, W, parts.T[:N_NODES])
    acc = _scatter(y, src2d, dst2d, jnp.zeros((N_PAD, DIM), jnp.float32))
    out = _combine(acc, y, disb, b.reshape(1, DIM))
    return out
